# Initial kernel scaffold; baseline (speedup 1.0000x reference)
import jax
import jax.numpy as jnp
from jax.experimental import pallas as pl


def kernel(h, edge_index, edge_attr, We, be, Wpre, bpre, Wpost, bpost, Wlin, blin, gamma, beta):
    raise NotImplementedError("write your pallas kernel here")


# speedup vs baseline: 1.1291x; 1.1291x over previous
import functools
import math

import jax
import jax.numpy as jnp
import numpy as np
from jax.experimental import pallas as pl

N = 10000
E = 320000
D = 128
AVG_LOG = float(np.log(33.0))

_NODE_BLK = 1000
_EDGE_BLK = 8000


def _gf_body(h_ref, w_ref, out_ref):
    out_ref[...] = jnp.dot(h_ref[...], w_ref[...],
                           preferred_element_type=jnp.float32)


def _node_gf(h, W12):
    return pl.pallas_call(
        _gf_body,
        grid=(N // _NODE_BLK,),
        in_specs=[
            pl.BlockSpec((_NODE_BLK, D), lambda i: (i, 0)),
            pl.BlockSpec((D, 2 * D), lambda i: (0, 0)),
        ],
        out_specs=pl.BlockSpec((_NODE_BLK, 2 * D), lambda i: (i, 0)),
        out_shape=jax.ShapeDtypeStruct((N, 2 * D), jnp.float32),
    )(h, W12)


def _ee_body(ea_ref, wc_ref, bc_ref, out_ref):
    out_ref[...] = jnp.dot(ea_ref[...], wc_ref[...],
                           preferred_element_type=jnp.float32) + bc_ref[...]


def _edge_ee(edge_attr, Wc, bc):
    ED = edge_attr.shape[1]
    return pl.pallas_call(
        _ee_body,
        grid=(E // _EDGE_BLK,),
        in_specs=[
            pl.BlockSpec((_EDGE_BLK, ED), lambda i: (i, 0)),
            pl.BlockSpec((ED, D), lambda i: (0, 0)),
            pl.BlockSpec((1, D), lambda i: (0, 0)),
        ],
        out_specs=pl.BlockSpec((_EDGE_BLK, D), lambda i: (i, 0)),
        out_shape=jax.ShapeDtypeStruct((E, D), jnp.float32),
    )(edge_attr, Wc, bc)


def _post_body(h_ref, g_ref, cnt_ref, sum_ref, sq_ref, mx_ref, mn_ref,
               wpost_ref, bpost_ref, wlin_ref, blin_ref, y_ref, stats_ref):
    i = pl.program_id(0)
    cnt = cnt_ref[...]
    deg = jnp.maximum(cnt, 1.0)
    g = g_ref[...]
    nonempty = cnt > 0.0
    mean = (cnt * g + sum_ref[...]) / deg
    mx = jnp.where(nonempty, g + mx_ref[...], 0.0)
    mn = jnp.where(nonempty, g + mn_ref[...], 0.0)
    meanq = sum_ref[...] / deg
    var = sq_ref[...] / deg - meanq * meanq
    std = jnp.sqrt(jnp.maximum(var, 0.0) + 1e-5)
    logd = jnp.log(deg + 1.0)
    amp = logd * (1.0 / AVG_LOG)
    att = AVG_LOG / logd
    agg = jnp.concatenate([mean, mn, mx, std], axis=-1)
    x = jnp.concatenate([h_ref[...], agg, agg * amp, agg * att], axis=-1)
    y = jnp.dot(x, wpost_ref[...], preferred_element_type=jnp.float32) \
        + bpost_ref[...]
    y = jnp.dot(y, wlin_ref[...], preferred_element_type=jnp.float32) \
        + blin_ref[...]
    y_ref[...] = y

    @pl.when(i == 0)
    def _():
        stats_ref[...] = jnp.zeros_like(stats_ref)

    s1 = jnp.sum(y, axis=0, keepdims=True)
    s2 = jnp.sum(y * y, axis=0, keepdims=True)
    stats_ref[...] += jnp.concatenate([s1, s2], axis=0)


def _post(h, g, cnt, sumq, sumq2, maxq, minq, Wpost, bpost, Wlin, blin):
    B = _NODE_BLK
    return pl.pallas_call(
        _post_body,
        grid=(N // B,),
        in_specs=[
            pl.BlockSpec((B, D), lambda i: (i, 0)),
            pl.BlockSpec((B, D), lambda i: (i, 0)),
            pl.BlockSpec((B, 1), lambda i: (i, 0)),
            pl.BlockSpec((B, D), lambda i: (i, 0)),
            pl.BlockSpec((B, D), lambda i: (i, 0)),
            pl.BlockSpec((B, D), lambda i: (i, 0)),
            pl.BlockSpec((B, D), lambda i: (i, 0)),
            pl.BlockSpec((13 * D, D), lambda i: (0, 0)),
            pl.BlockSpec((1, D), lambda i: (0, 0)),
            pl.BlockSpec((D, D), lambda i: (0, 0)),
            pl.BlockSpec((1, D), lambda i: (0, 0)),
        ],
        out_specs=[
            pl.BlockSpec((B, D), lambda i: (i, 0)),
            pl.BlockSpec((2, D), lambda i: (0, 0)),
        ],
        out_shape=[
            jax.ShapeDtypeStruct((N, D), jnp.float32),
            jax.ShapeDtypeStruct((2, D), jnp.float32),
        ],
    )(h, g, cnt, sumq, sumq2, maxq, minq, Wpost, bpost, Wlin, blin)


def _bn_body(y_ref, stats_ref, gamma_ref, beta_ref, h_ref, out_ref):
    mu = stats_ref[0:1, :] * (1.0 / N)
    ex2 = stats_ref[1:2, :] * (1.0 / N)
    v = ex2 - mu * mu
    inv = jax.lax.rsqrt(v + 1e-5)
    y = (y_ref[...] - mu) * inv * gamma_ref[...] + beta_ref[...]
    out_ref[...] = jnp.maximum(y, 0.0) + h_ref[...]


def _bn(y, stats, gamma, beta, h):
    B = _NODE_BLK
    return pl.pallas_call(
        _bn_body,
        grid=(N // B,),
        in_specs=[
            pl.BlockSpec((B, D), lambda i: (i, 0)),
            pl.BlockSpec((2, D), lambda i: (0, 0)),
            pl.BlockSpec((1, D), lambda i: (0, 0)),
            pl.BlockSpec((1, D), lambda i: (0, 0)),
            pl.BlockSpec((B, D), lambda i: (i, 0)),
        ],
        out_specs=pl.BlockSpec((B, D), lambda i: (i, 0)),
        out_shape=jax.ShapeDtypeStruct((N, D), jnp.float32),
    )(y, stats, gamma, beta, h)


def kernel(h, edge_index, edge_attr, We, be, Wpre, bpre, Wpost, bpost,
           Wlin, blin, gamma, beta):
    W1 = Wpre[:D]
    W2 = Wpre[D:2 * D]
    W3 = Wpre[2 * D:]
    Wc = We @ W3
    bc = (be @ W3 + bpre)[None, :]

    gf = _node_gf(h, jnp.concatenate([W1, W2], axis=1))
    g = gf[:, :D]
    f = gf[:, D:]
    ee = _edge_ee(edge_attr, Wc, bc)

    src = edge_index[0]
    dst = edge_index[1]
    q = jnp.take(f, src, axis=0) + ee
    ones = jnp.ones((E,), jnp.float32)
    cnt = jax.ops.segment_sum(ones, dst, num_segments=N)[:, None]
    sumq = jax.ops.segment_sum(q, dst, num_segments=N)
    sumq2 = jax.ops.segment_sum(q * q, dst, num_segments=N)
    maxq = jax.ops.segment_max(q, dst, num_segments=N)
    minq = jax.ops.segment_min(q, dst, num_segments=N)

    y, stats = _post(h, g, cnt, sumq, sumq2, maxq, minq,
                     Wpost, bpost[None, :], Wlin, blin[None, :])
    return _bn(y, stats, gamma[None, :], beta[None, :], h)
